# Initial kernel scaffold; baseline (speedup 1.0000x reference)
import jax
import jax.numpy as jnp
from jax.experimental import pallas as pl


def kernel(x0, x1, x2, pos, params, edge_index_0, edge_index_1, edge_index_2, batch):
    raise NotImplementedError("write your pallas kernel here")


# speedup vs baseline: 1.4772x; 1.4772x over previous
import functools
import jax
import jax.numpy as jnp
from jax import lax
from jax.experimental import pallas as pl
from jax.experimental.pallas import tpu as pltpu
from jax.experimental.pallas import tpu_sc as plsc

N = 10000
E = 320000
H = 128
NG = 16

_BN = 400
_CE_S = 12800
_CE_I = 2560
_CHF = 4000
_GP = 128
_INV_CH = 1280
_K = 128


def _tile_edges(cid, sid):
    wid = sid + cid
    start = (156 * wid + jnp.minimum(wid, 4)) * _GP
    ngroups = 156 + jnp.where(wid < 4, 1, 0)
    return start, ngroups


def _inv_body(pos_hbm, s0_hbm, d0_hbm, s1_hbm, d1_hbm, s2_hbm, d2_hbm,
              o0_hbm, o1_hbm, o2_hbm, src_v, dst_v, ps_v, pd_v, dd_v, sem):
    cid = lax.axis_index("c")
    sid = lax.axis_index("s")
    start, ngroups = _tile_edges(cid, sid)

    def do_chunk(s_hbm, d_hbm, o_hbm, eb, ng):
        npos = ng * _GP
        pltpu.sync_copy(s_hbm.at[pl.ds(eb, npos)], src_v.at[pl.ds(0, npos)])
        pltpu.sync_copy(d_hbm.at[pl.ds(eb, npos)], dst_v.at[pl.ds(0, npos)])
        for j in range(ng):
            sl = pl.ds(j * _GP, _GP)
            cs = pltpu.async_copy(pos_hbm.at[src_v.at[sl]], ps_v, sem)
            cd = pltpu.async_copy(pos_hbm.at[dst_v.at[sl]], pd_v, sem)
            cs.wait()
            cd.wait()

            def sub_row(r, _):
                dd_v[r, pl.ds(0, 16)] = (
                    pd_v[r, pl.ds(0, 16)] - ps_v[r, pl.ds(0, 16)])
                return 0
            lax.fori_loop(0, _GP, sub_row, 0)
            pltpu.sync_copy(dd_v, o_hbm.at[pl.ds(eb + j * _GP, _GP)])

    nfull = ngroups // 10
    grem = ngroups - nfull * 10
    for s_hbm, d_hbm, o_hbm in ((s0_hbm, d0_hbm, o0_hbm),
                                (s1_hbm, d1_hbm, o1_hbm),
                                (s2_hbm, d2_hbm, o2_hbm)):
        def full_chunk(ch, _):
            do_chunk(s_hbm, d_hbm, o_hbm, start + ch * _INV_CH, 10)
            return 0
        lax.fori_loop(0, nfull, full_chunk, 0)

        def rem_chunk(g, _):
            do_chunk(s_hbm, d_hbm, o_hbm, start + nfull * _INV_CH + g * _GP, 1)
            return 0
        lax.fori_loop(0, grem, rem_chunk, 0)


def _mp_body(y_hbm, ie_hbm, s_hbm, d_hbm, out_hbm, agg_s, src_v, dst_v,
             g_v, ie_v, sem):
    cid = lax.axis_index("c")
    sid = lax.axis_index("s")
    start, ngroups = _tile_edges(cid, sid)
    rbase = sid * 640
    nrows = jnp.where(sid == 15, 5, 8)
    zz = jnp.zeros((16,), jnp.float32)

    def zrow(r, _):
        for c in range(8):
            g_v[r, pl.ds(c * 16, 16)] = zz
        return 0
    lax.fori_loop(0, _K, zrow, 0)

    def zcopy(k, _):
        pltpu.sync_copy(g_v.at[pl.ds(0, 80)], agg_s.at[pl.ds(rbase + k * 80, 80)])
        return 0
    lax.fori_loop(0, nrows, zcopy, 0)
    plsc.subcore_barrier()

    def chunk(ch, _):
        eb = start + ch * _K
        pltpu.sync_copy(s_hbm.at[pl.ds(eb, _K)], src_v)
        pltpu.sync_copy(d_hbm.at[pl.ds(eb, _K)], dst_v)
        pltpu.async_copy(y_hbm.at[src_v], g_v, sem).wait()
        pltpu.sync_copy(ie_hbm.at[pl.ds(eb, _K)], ie_v)

        def row(r, _):
            for c in range(8):
                sl = pl.ds(c * 16, 16)
                t = g_v[r, sl] + ie_v[r, sl]
                g_v[r, sl] = t / (1.0 + jnp.exp(-t))
            return 0
        lax.fori_loop(0, _K, row, 0)
        pltpu.sync_copy(g_v, agg_s.at[dst_v], add=True)
        return 0
    lax.fori_loop(0, ngroups, chunk, 0)
    plsc.subcore_barrier()

    def dump(k, _):
        sl = pl.ds(rbase + k * 80, 80)
        pltpu.sync_copy(agg_s.at[sl], out_hbm.at[sl])
        return 0
    lax.fori_loop(0, nrows, dump, 0)


@functools.cache
def _sc_kernels():
    mesh = plsc.VectorSubcoreMesh(core_axis_name="c", subcore_axis_name="s", num_cores=1)
    inv = pl.kernel(
        _inv_body,
        out_type=[jax.ShapeDtypeStruct((E, 16), jnp.float32)] * 3,
        mesh=mesh,
        scratch_types=[
            pltpu.VMEM((_INV_CH,), jnp.int32),
            pltpu.VMEM((_INV_CH,), jnp.int32),
            pltpu.VMEM((_GP, H), jnp.float32),
            pltpu.VMEM((_GP, H), jnp.float32),
            pltpu.VMEM((_GP, 16), jnp.float32),
            pltpu.SemaphoreType.DMA,
        ],
    )
    mp = pl.kernel(
        _mp_body,
        out_type=jax.ShapeDtypeStruct((N, H), jnp.float32),
        mesh=mesh,
        scratch_types=[
            pltpu.VMEM_SHARED((N, H), jnp.float32),
            pltpu.VMEM((_K,), jnp.int32),
            pltpu.VMEM((_K,), jnp.int32),
            pltpu.VMEM((_K, H), jnp.float32),
            pltpu.VMEM((_K, H), jnp.float32),
            pltpu.SemaphoreType.DMA,
        ],
    )
    return inv, mp


def _silu(t):
    return t / (1.0 + jnp.exp(-t))


def _mm_bias_kernel(x_ref, w_ref, b_ref, o_ref):
    o_ref[...] = jnp.dot(x_ref[...], w_ref[...],
                         preferred_element_type=jnp.float32) + b_ref[...]


def _mm_bias(x, w, b):
    return pl.pallas_call(
        _mm_bias_kernel,
        grid=(N // _BN,),
        in_specs=[pl.BlockSpec((_BN, H), lambda i: (i, 0)),
                  pl.BlockSpec((H, H), lambda i: (0, 0)),
                  pl.BlockSpec((1, H), lambda i: (0, 0))],
        out_specs=pl.BlockSpec((_BN, H), lambda i: (i, 0)),
        out_shape=jax.ShapeDtypeStruct((N, H), jnp.float32),
    )(x, w, b)


def _f_kernel(d_ref, o_ref):
    d = d_ref[...]
    dx = d[:, 0:1]
    dy = d[:, 1:2]
    dz = d[:, 2:3]
    n2 = dx * dx + dy * dy + dz * dz
    zer = jnp.zeros_like(n2)
    o_ref[...] = jnp.concatenate(
        [jnp.sqrt(n2), jnp.abs(dx), jnp.abs(dy), jnp.abs(dz), n2,
         zer, zer, zer], axis=1)


def _f_mm(d):
    return pl.pallas_call(
        _f_kernel,
        grid=(E // _CHF,),
        in_specs=[pl.BlockSpec((_CHF, 16), lambda i: (i, 0))],
        out_specs=pl.BlockSpec((_CHF, 8), lambda i: (i, 0)),
        out_shape=jax.ShapeDtypeStruct((E, 8), jnp.float32),
    )(d)


def _stats_kernel(f_ref, o_ref):
    i = pl.program_id(0)
    fm = f_ref[...]
    su = jnp.sum(fm, axis=0, keepdims=True)
    ss = jnp.sum(fm * fm, axis=0, keepdims=True)
    blk = jnp.concatenate([su, ss], axis=0)

    @pl.when(i == 0)
    def _():
        o_ref[...] = blk

    @pl.when(i > 0)
    def _():
        o_ref[...] = o_ref[...] + blk

    @pl.when(i == pl.num_programs(0) - 1)
    def _():
        tot = o_ref[...]
        mu = tot[0:1, :] / E
        var = tot[1:2, :] / E - mu * mu
        o_ref[...] = jnp.concatenate([mu, var], axis=0)


def _stats(f):
    return pl.pallas_call(
        _stats_kernel,
        grid=(E // _CE_S,),
        in_specs=[pl.BlockSpec((_CE_S, 8), lambda i: (i, 0))],
        out_specs=pl.BlockSpec((2, 8), lambda i: (0, 0)),
        out_shape=jax.ShapeDtypeStruct((2, 8), jnp.float32),
    )(f)


def _y_kernel(x_ref, w_ref, b_ref, st_ref, o_ref):
    wmx = w_ref[0:H, :]
    wmi = w_ref[H:H + 8, :]
    st = st_ref[...]
    ratio = st[0:1, :] / jnp.sqrt(st[1:2, :] + 1e-5)
    shift = jnp.dot(ratio, wmi, preferred_element_type=jnp.float32)
    o_ref[...] = (jnp.dot(x_ref[...], wmx, preferred_element_type=jnp.float32)
                  + b_ref[...] - shift)


def _y_mm(x, wm_pad, b, st):
    return pl.pallas_call(
        _y_kernel,
        grid=(N // _BN,),
        in_specs=[pl.BlockSpec((_BN, H), lambda i: (i, 0)),
                  pl.BlockSpec((H + 8, H), lambda i: (0, 0)),
                  pl.BlockSpec((1, H), lambda i: (0, 0)),
                  pl.BlockSpec((2, 8), lambda i: (0, 0))],
        out_specs=pl.BlockSpec((_BN, H), lambda i: (i, 0)),
        out_shape=jax.ShapeDtypeStruct((N, H), jnp.float32),
    )(x, wm_pad, b, st)


def _ie_kernel(f_ref, wmi_ref, st_ref, o_ref):
    st = st_ref[...]
    rs = 1.0 / jnp.sqrt(st[1:2, :] + 1e-5)
    fn = f_ref[...] * rs
    o_ref[...] = jnp.dot(fn, wmi_ref[...], preferred_element_type=jnp.float32)


def _ie_mm(f, wmi, st):
    return pl.pallas_call(
        _ie_kernel,
        grid=(E // _CE_I,),
        in_specs=[pl.BlockSpec((_CE_I, 8), lambda i: (i, 0)),
                  pl.BlockSpec((8, H), lambda i: (0, 0)),
                  pl.BlockSpec((2, 8), lambda i: (0, 0))],
        out_specs=pl.BlockSpec((_CE_I, H), lambda i: (i, 0)),
        out_shape=jax.ShapeDtypeStruct((E, H), jnp.float32),
    )(f, wmi, st)


def _upd_res_kernel(p_ref, x_ref, w_ref, b_ref, o_ref):
    a = p_ref[...]
    h = jnp.dot(a, w_ref[...], preferred_element_type=jnp.float32) + b_ref[...]
    o_ref[...] = x_ref[...] + _silu(h)


def _upd_cross_kernel(p_ref, w_ref, b_ref, o_ref):
    a = p_ref[...]
    h = jnp.dot(a, w_ref[...], preferred_element_type=jnp.float32) + b_ref[...]
    o_ref[...] = _silu(h)


def _update(partials, x, w, b, residual):
    pspec = pl.BlockSpec((_BN, H), lambda i: (i, 0))
    wspec = pl.BlockSpec((H, H), lambda i: (0, 0))
    bspec = pl.BlockSpec((1, H), lambda i: (0, 0))
    ospec = pl.BlockSpec((_BN, H), lambda i: (i, 0))
    oshape = jax.ShapeDtypeStruct((N, H), jnp.float32)
    if residual:
        return pl.pallas_call(
            _upd_res_kernel, grid=(N // _BN,),
            in_specs=[pspec, ospec, wspec, bspec],
            out_specs=ospec, out_shape=oshape,
        )(partials, x, w, b)
    return pl.pallas_call(
        _upd_cross_kernel, grid=(N // _BN,),
        in_specs=[pspec, wspec, bspec],
        out_specs=ospec, out_shape=oshape,
    )(partials, w, b)


def _pool_kernel(b_ref, xa_ref, xb_ref, xc_ref, oa_ref, ob_ref, oc_ref):
    i = pl.program_id(0)
    b = b_ref[0]
    seg = lax.broadcasted_iota(jnp.int32, (NG, 1), 0)
    oh = (b == seg).astype(jnp.float32)
    for xr, orf in ((xa_ref, oa_ref), (xb_ref, ob_ref), (xc_ref, oc_ref)):
        blk = lax.dot_general(oh, xr[...], (((1,), (0,)), ((), ())),
                              preferred_element_type=jnp.float32)

        @pl.when(i == 0)
        def _():
            orf[...] = blk

        @pl.when(i > 0)
        def _():
            orf[...] = orf[...] + blk


def _pool(batch3, xa, xb, xc):
    xspec = pl.BlockSpec((_BN, H), lambda i: (i, 0))
    ospec = pl.BlockSpec((NG, H), lambda i: (0, 0))
    oshape = jax.ShapeDtypeStruct((NG, H), jnp.float32)
    return pl.pallas_call(
        _pool_kernel,
        grid=(N // _BN,),
        in_specs=[pl.BlockSpec((1, 1, _BN), lambda i: (i, 0, 0)),
                  xspec, xspec, xspec],
        out_specs=[ospec, ospec, ospec],
        out_shape=[oshape, oshape, oshape],
    )(batch3, xa, xb, xc)


def _head_kernel(pa_ref, pb_ref, pc_ref, w1_ref, b1_ref, w2_ref, b2_ref, o_ref):
    h = (jnp.dot(pa_ref[...], w1_ref[0:H, :], preferred_element_type=jnp.float32)
         + jnp.dot(pb_ref[...], w1_ref[H:2 * H, :], preferred_element_type=jnp.float32)
         + jnp.dot(pc_ref[...], w1_ref[2 * H:3 * H, :], preferred_element_type=jnp.float32)
         + b1_ref[...])
    h = _silu(h)
    o_ref[...] = jnp.dot(h, w2_ref[...], preferred_element_type=jnp.float32) + b2_ref[...]


def _head(pa, pb, pc, w1, b1, w2p, b2p):
    full = lambda shape: pl.BlockSpec(shape, lambda: tuple(0 for _ in shape))
    return pl.pallas_call(
        _head_kernel,
        in_specs=[full((NG, H)), full((NG, H)), full((NG, H)),
                  full((3 * H, H)), full((1, H)), full((H, 8)), full((1, 8))],
        out_specs=full((NG, 8)),
        out_shape=jax.ShapeDtypeStruct((NG, 8), jnp.float32),
    )(pa, pb, pc, w1, b1, w2p, b2p)


def kernel(x0, x1, x2, pos, params, edge_index_0, edge_index_1, edge_index_2, batch):
    del x1, x2
    e0 = edge_index_0.astype(jnp.int32)
    e1 = edge_index_1.astype(jnp.int32)
    e2 = edge_index_2.astype(jnp.int32)
    pos = pos.astype(jnp.float32)
    batch3 = batch.astype(jnp.int32).reshape(N // _BN, 1, _BN)
    srcs = {0: e0[0], 1: e1[0], 2: e2[0]}
    dsts = {0: e0[1], 1: e1[1], 2: e2[1]}

    pos128 = jnp.pad(pos, ((0, 0), (0, H - 3)))
    inv_call, mp_call = _sc_kernels()
    dd0, dd1, dd2 = inv_call(
        pos128, srcs[0], dsts[0], srcs[1], dsts[1], srcs[2], dsts[2])
    fs = {0: _f_mm(dd0), 1: _f_mm(dd1), 2: _f_mm(dd2)}
    st = {0: _stats(fs[0]), 1: _stats(fs[1]), 2: _stats(fs[2])}

    def mp(x, r, p, wkey, bkey, residual):
        wm = jnp.pad(p['Wm'], ((0, 3), (0, 0)))
        y = _y_mm(x, wm, p['bm'][None, :], st[r])
        ie = _ie_mm(fs[r], wm[H:H + 8], st[r])
        partials = mp_call(y, ie, srcs[r], dsts[r])
        return _update(partials, x, p[wkey], p[bkey][None, :], residual)

    pe = params['emb']['0']
    x = _mm_bias(x0, pe['W'], pe['b'][None, :])
    for l in range(2):
        x = mp(x, 0, params['intra']['0_' + str(l)], 'Wu', 'bu', True)
    x1h = mp(x, 1, params['cross']['0_1'], 'Wo', 'bo', False)
    for l in range(2):
        x1h = mp(x1h, 1, params['intra']['1_' + str(l)], 'Wu', 'bu', True)
    x2h = mp(x1h, 2, params['cross']['1_2'], 'Wo', 'bo', False)
    for l in range(2):
        x2h = mp(x2h, 2, params['intra']['2_' + str(l)], 'Wu', 'bu', True)

    pa, pb, pc = _pool(batch3, x, x1h, x2h)
    pp = params['pool']
    w2p = jnp.pad(pp['W2'], ((0, 0), (0, 7)))
    b2p = jnp.pad(pp['b2'], (0, 7))[None, :]
    out = _head(pa, pb, pc, pp['W1'], pp['b1'][None, :], w2p, b2p)
    return out[:, 0:1]


# speedup vs baseline: 2.1780x; 1.4744x over previous
import functools
import jax
import jax.numpy as jnp
from jax import lax
from jax.experimental import pallas as pl
from jax.experimental.pallas import tpu as pltpu
from jax.experimental.pallas import tpu_sc as plsc

N = 10000
E = 320000
H = 128
NG = 16

_BN = 400
_CE_S = 12800
_CE_I = 2560
_CHF = 4000
_GP = 128
_INV_CH = 1280
_K = 64


def _tile_edges(cid, sid):
    wid = sid + cid
    start = (156 * wid + jnp.minimum(wid, 4)) * _GP
    ngroups = 156 + jnp.where(wid < 4, 1, 0)
    return start, ngroups


def _inv_body(pos_hbm, s0_hbm, d0_hbm, s1_hbm, d1_hbm, s2_hbm, d2_hbm,
              o0_hbm, o1_hbm, o2_hbm,
              sa_v, da_v, sb_v, db_v, psa_v, pda_v, psb_v, pdb_v,
              dda_v, ddb_v, sema, semb, semia, semib, semoa, semob):
    cid = lax.axis_index("c")
    sid = lax.axis_index("s")
    start, ngroups = _tile_edges(cid, sid)
    wid = sid + cid

    def compute(ps_v, pd_v, dd_v):
        def sub_row(r, _):
            dd_v[r, pl.ds(0, 16)] = pd_v[r, pl.ds(0, 16)] - ps_v[r, pl.ds(0, 16)]
            return 0
        lax.fori_loop(0, _GP, sub_row, 0)

    for s_hbm, d_hbm, o_hbm in ((s0_hbm, d0_hbm, o0_hbm),
                                (s1_hbm, d1_hbm, o1_hbm),
                                (s2_hbm, d2_hbm, o2_hbm)):
        def eb(j):
            return start + j * _GP

        pltpu.sync_copy(s_hbm.at[pl.ds(start, _GP)], sa_v)
        pltpu.sync_copy(d_hbm.at[pl.ds(start, _GP)], da_v)
        pltpu.async_copy(pos_hbm.at[sa_v], psa_v, sema)
        pltpu.async_copy(pos_hbm.at[da_v], pda_v, sema)
        pltpu.async_copy(s_hbm.at[pl.ds(eb(1), _GP)], sb_v, semib)
        pltpu.async_copy(d_hbm.at[pl.ds(eb(1), _GP)], db_v, semib)

        def pair(p, _):
            j = p * 2
            pltpu.make_async_copy(pos_hbm.at[sa_v], psa_v, sema).wait()
            pltpu.make_async_copy(pos_hbm.at[da_v], pda_v, sema).wait()
            pltpu.make_async_copy(s_hbm.at[pl.ds(eb(j + 1), _GP)], sb_v, semib).wait()
            pltpu.make_async_copy(d_hbm.at[pl.ds(eb(j + 1), _GP)], db_v, semib).wait()
            pltpu.async_copy(pos_hbm.at[sb_v], psb_v, semb)
            pltpu.async_copy(pos_hbm.at[db_v], pdb_v, semb)

            @pl.when(p > 0)
            def _():
                pltpu.make_async_copy(dda_v, o_hbm.at[pl.ds(eb(j - 2), _GP)], semoa).wait()
            compute(psa_v, pda_v, dda_v)
            pltpu.async_copy(dda_v, o_hbm.at[pl.ds(eb(j), _GP)], semoa)
            pltpu.async_copy(s_hbm.at[pl.ds(eb(j + 2), _GP)], sa_v, semia)
            pltpu.async_copy(d_hbm.at[pl.ds(eb(j + 2), _GP)], da_v, semia)
            pltpu.make_async_copy(pos_hbm.at[sb_v], psb_v, semb).wait()
            pltpu.make_async_copy(pos_hbm.at[db_v], pdb_v, semb).wait()
            pltpu.make_async_copy(s_hbm.at[pl.ds(eb(j + 2), _GP)], sa_v, semia).wait()
            pltpu.make_async_copy(d_hbm.at[pl.ds(eb(j + 2), _GP)], da_v, semia).wait()

            @pl.when((p < 77) | (wid < 4))
            def _():
                pltpu.async_copy(pos_hbm.at[sa_v], psa_v, sema)
                pltpu.async_copy(pos_hbm.at[da_v], pda_v, sema)

            @pl.when(p > 0)
            def _():
                pltpu.make_async_copy(ddb_v, o_hbm.at[pl.ds(eb(j - 1), _GP)], semob).wait()
            compute(psb_v, pdb_v, ddb_v)
            pltpu.async_copy(ddb_v, o_hbm.at[pl.ds(eb(j + 1), _GP)], semob)
            pltpu.async_copy(s_hbm.at[pl.ds(eb(j + 3), _GP)], sb_v, semib)
            pltpu.async_copy(d_hbm.at[pl.ds(eb(j + 3), _GP)], db_v, semib)
            return 0
        lax.fori_loop(0, 78, pair, 0)

        pltpu.make_async_copy(s_hbm.at[pl.ds(eb(157), _GP)], sb_v, semib).wait()
        pltpu.make_async_copy(d_hbm.at[pl.ds(eb(157), _GP)], db_v, semib).wait()

        @pl.when(wid < 4)
        def _():
            pltpu.make_async_copy(pos_hbm.at[sa_v], psa_v, sema).wait()
            pltpu.make_async_copy(pos_hbm.at[da_v], pda_v, sema).wait()
            pltpu.make_async_copy(dda_v, o_hbm.at[pl.ds(eb(154), _GP)], semoa).wait()
            compute(psa_v, pda_v, dda_v)
            pltpu.async_copy(dda_v, o_hbm.at[pl.ds(eb(156), _GP)], semoa)
        pltpu.make_async_copy(dda_v, o_hbm.at[pl.ds(start, _GP)], semoa).wait()
        pltpu.make_async_copy(ddb_v, o_hbm.at[pl.ds(start, _GP)], semob).wait()


def _mp_body(y_hbm, ie_hbm, s_hbm, d_hbm, out_hbm, agg_s,
             sa_v, da_v, sb_v, db_v, ga_v, gb_v, iea_v, ieb_v,
             sema, semb, semia, semib):
    cid = lax.axis_index("c")
    sid = lax.axis_index("s")
    start, ngroups = _tile_edges(cid, sid)
    wid = sid + cid
    rbase = sid * 640
    nrows = jnp.where(sid == 15, 5, 8)
    zz = jnp.zeros((16,), jnp.float32)

    def zrow(r, _):
        for c in range(8):
            ga_v[r, pl.ds(c * 16, 16)] = zz
        return 0
    lax.fori_loop(0, _K, zrow, 0)

    def zcopy(k, _):
        pltpu.sync_copy(ga_v.at[pl.ds(0, 40)], agg_s.at[pl.ds(rbase + k * 40, 40)])
        return 0
    lax.fori_loop(0, nrows * 2, zcopy, 0)
    plsc.subcore_barrier()

    def eb(j):
        return start + j * _K

    def compute(g_v, ie_v):
        def row(r, _):
            for c in range(8):
                sl = pl.ds(c * 16, 16)
                t = g_v[r, sl] + ie_v[r, sl]
                g_v[r, sl] = t / (1.0 + jnp.exp(-t))
            return 0
        lax.fori_loop(0, _K, row, 0)

    pltpu.sync_copy(s_hbm.at[pl.ds(start, _K)], sa_v)
    pltpu.sync_copy(d_hbm.at[pl.ds(start, _K)], da_v)
    pltpu.async_copy(y_hbm.at[sa_v], ga_v, sema)
    pltpu.async_copy(ie_hbm.at[pl.ds(start, _K)], iea_v, sema)
    pltpu.async_copy(s_hbm.at[pl.ds(eb(1), _K)], sb_v, semib)
    pltpu.async_copy(d_hbm.at[pl.ds(eb(1), _K)], db_v, semib)

    def pair(p, _):
        j = p * 2
        pltpu.make_async_copy(y_hbm.at[sa_v], ga_v, sema).wait()
        pltpu.make_async_copy(ie_hbm.at[pl.ds(eb(j), _K)], iea_v, sema).wait()
        pltpu.make_async_copy(s_hbm.at[pl.ds(eb(j + 1), _K)], sb_v, semib).wait()
        pltpu.make_async_copy(d_hbm.at[pl.ds(eb(j + 1), _K)], db_v, semib).wait()
        pltpu.async_copy(y_hbm.at[sb_v], gb_v, semb)
        pltpu.async_copy(ie_hbm.at[pl.ds(eb(j + 1), _K)], ieb_v, semb)
        compute(ga_v, iea_v)
        pltpu.sync_copy(ga_v, agg_s.at[da_v], add=True)
        pltpu.async_copy(s_hbm.at[pl.ds(eb(j + 2), _K)], sa_v, semia)
        pltpu.async_copy(d_hbm.at[pl.ds(eb(j + 2), _K)], da_v, semia)
        pltpu.make_async_copy(y_hbm.at[sb_v], gb_v, semb).wait()
        pltpu.make_async_copy(ie_hbm.at[pl.ds(eb(j + 1), _K)], ieb_v, semb).wait()
        pltpu.make_async_copy(s_hbm.at[pl.ds(eb(j + 2), _K)], sa_v, semia).wait()
        pltpu.make_async_copy(d_hbm.at[pl.ds(eb(j + 2), _K)], da_v, semia).wait()

        @pl.when((p < 155) | (wid < 4))
        def _():
            pltpu.async_copy(y_hbm.at[sa_v], ga_v, sema)
            pltpu.async_copy(ie_hbm.at[pl.ds(eb(j + 2), _K)], iea_v, sema)
        compute(gb_v, ieb_v)
        pltpu.sync_copy(gb_v, agg_s.at[db_v], add=True)
        pltpu.async_copy(s_hbm.at[pl.ds(eb(j + 3), _K)], sb_v, semib)
        pltpu.async_copy(d_hbm.at[pl.ds(eb(j + 3), _K)], db_v, semib)
        return 0
    lax.fori_loop(0, 156, pair, 0)

    pltpu.make_async_copy(s_hbm.at[pl.ds(eb(313), _K)], sb_v, semib).wait()
    pltpu.make_async_copy(d_hbm.at[pl.ds(eb(313), _K)], db_v, semib).wait()

    @pl.when(wid < 4)
    def _():
        pltpu.async_copy(y_hbm.at[sb_v], gb_v, semb)
        pltpu.async_copy(ie_hbm.at[pl.ds(eb(313), _K)], ieb_v, semb)
        pltpu.make_async_copy(y_hbm.at[sa_v], ga_v, sema).wait()
        pltpu.make_async_copy(ie_hbm.at[pl.ds(eb(312), _K)], iea_v, sema).wait()
        compute(ga_v, iea_v)
        pltpu.sync_copy(ga_v, agg_s.at[da_v], add=True)
        pltpu.make_async_copy(y_hbm.at[sb_v], gb_v, semb).wait()
        pltpu.make_async_copy(ie_hbm.at[pl.ds(eb(313), _K)], ieb_v, semb).wait()
        compute(gb_v, ieb_v)
        pltpu.sync_copy(gb_v, agg_s.at[db_v], add=True)
    plsc.subcore_barrier()

    def dump(k, _):
        sl = pl.ds(rbase + k * 80, 80)
        pltpu.sync_copy(agg_s.at[sl], out_hbm.at[sl])
        return 0
    lax.fori_loop(0, nrows, dump, 0)


@functools.cache
def _sc_kernels():
    mesh = plsc.VectorSubcoreMesh(core_axis_name="c", subcore_axis_name="s", num_cores=1)
    inv = pl.kernel(
        _inv_body,
        out_type=[jax.ShapeDtypeStruct((E, 16), jnp.float32)] * 3,
        mesh=mesh,
        scratch_types=[
            pltpu.VMEM((_GP,), jnp.int32),
            pltpu.VMEM((_GP,), jnp.int32),
            pltpu.VMEM((_GP,), jnp.int32),
            pltpu.VMEM((_GP,), jnp.int32),
            pltpu.VMEM((_GP, H), jnp.float32),
            pltpu.VMEM((_GP, H), jnp.float32),
            pltpu.VMEM((_GP, H), jnp.float32),
            pltpu.VMEM((_GP, H), jnp.float32),
            pltpu.VMEM((_GP, 16), jnp.float32),
            pltpu.VMEM((_GP, 16), jnp.float32),
            pltpu.SemaphoreType.DMA,
            pltpu.SemaphoreType.DMA,
            pltpu.SemaphoreType.DMA,
            pltpu.SemaphoreType.DMA,
            pltpu.SemaphoreType.DMA,
            pltpu.SemaphoreType.DMA,
        ],
    )
    mp = pl.kernel(
        _mp_body,
        out_type=jax.ShapeDtypeStruct((N, H), jnp.float32),
        mesh=mesh,
        scratch_types=[
            pltpu.VMEM_SHARED((N, H), jnp.float32),
            pltpu.VMEM((_K,), jnp.int32),
            pltpu.VMEM((_K,), jnp.int32),
            pltpu.VMEM((_K,), jnp.int32),
            pltpu.VMEM((_K,), jnp.int32),
            pltpu.VMEM((_K, H), jnp.float32),
            pltpu.VMEM((_K, H), jnp.float32),
            pltpu.VMEM((_K, H), jnp.float32),
            pltpu.VMEM((_K, H), jnp.float32),
            pltpu.SemaphoreType.DMA,
            pltpu.SemaphoreType.DMA,
            pltpu.SemaphoreType.DMA,
            pltpu.SemaphoreType.DMA,
        ],
    )
    return inv, mp


def _silu(t):
    return t / (1.0 + jnp.exp(-t))


def _mm_bias_kernel(x_ref, w_ref, b_ref, o_ref):
    o_ref[...] = jnp.dot(x_ref[...], w_ref[...],
                         preferred_element_type=jnp.float32,
                 precision=lax.Precision.HIGHEST) + b_ref[...]


def _mm_bias(x, w, b):
    return pl.pallas_call(
        _mm_bias_kernel,
        grid=(N // _BN,),
        in_specs=[pl.BlockSpec((_BN, H), lambda i: (i, 0)),
                  pl.BlockSpec((H, H), lambda i: (0, 0)),
                  pl.BlockSpec((1, H), lambda i: (0, 0))],
        out_specs=pl.BlockSpec((_BN, H), lambda i: (i, 0)),
        out_shape=jax.ShapeDtypeStruct((N, H), jnp.float32),
    )(x, w, b)


def _f_kernel(d_ref, o_ref):
    d = d_ref[...]
    dx = d[:, 0:1]
    dy = d[:, 1:2]
    dz = d[:, 2:3]
    n2 = dx * dx + dy * dy + dz * dz
    zer = jnp.zeros_like(n2)
    o_ref[...] = jnp.concatenate(
        [jnp.sqrt(n2), jnp.abs(dx), jnp.abs(dy), jnp.abs(dz), n2,
         zer, zer, zer], axis=1)


def _f_mm(d):
    return pl.pallas_call(
        _f_kernel,
        grid=(E // _CHF,),
        in_specs=[pl.BlockSpec((_CHF, 16), lambda i: (i, 0))],
        out_specs=pl.BlockSpec((_CHF, 8), lambda i: (i, 0)),
        out_shape=jax.ShapeDtypeStruct((E, 8), jnp.float32),
    )(d)


def _stats_kernel(f_ref, o_ref):
    i = pl.program_id(0)
    fm = f_ref[...]
    su = jnp.sum(fm, axis=0, keepdims=True)
    ss = jnp.sum(fm * fm, axis=0, keepdims=True)
    blk = jnp.concatenate([su, ss], axis=0)

    @pl.when(i == 0)
    def _():
        o_ref[...] = blk

    @pl.when(i > 0)
    def _():
        o_ref[...] = o_ref[...] + blk

    @pl.when(i == pl.num_programs(0) - 1)
    def _():
        tot = o_ref[...]
        mu = tot[0:1, :] / E
        var = tot[1:2, :] / E - mu * mu
        o_ref[...] = jnp.concatenate([mu, var], axis=0)


def _stats(f):
    return pl.pallas_call(
        _stats_kernel,
        grid=(E // _CE_S,),
        in_specs=[pl.BlockSpec((_CE_S, 8), lambda i: (i, 0))],
        out_specs=pl.BlockSpec((2, 8), lambda i: (0, 0)),
        out_shape=jax.ShapeDtypeStruct((2, 8), jnp.float32),
    )(f)


def _y_kernel(x_ref, w_ref, b_ref, st_ref, o_ref):
    wmx = w_ref[0:H, :]
    wmi = w_ref[H:H + 8, :]
    st = st_ref[...]
    ratio = st[0:1, :] / jnp.sqrt(st[1:2, :] + 1e-5)
    shift = jnp.dot(ratio, wmi, preferred_element_type=jnp.float32,
                 precision=lax.Precision.HIGHEST)
    o_ref[...] = (jnp.dot(x_ref[...], wmx, preferred_element_type=jnp.float32,
                 precision=lax.Precision.HIGHEST)
                  + b_ref[...] - shift)


def _y_mm(x, wm_pad, b, st):
    return pl.pallas_call(
        _y_kernel,
        grid=(N // _BN,),
        in_specs=[pl.BlockSpec((_BN, H), lambda i: (i, 0)),
                  pl.BlockSpec((H + 8, H), lambda i: (0, 0)),
                  pl.BlockSpec((1, H), lambda i: (0, 0)),
                  pl.BlockSpec((2, 8), lambda i: (0, 0))],
        out_specs=pl.BlockSpec((_BN, H), lambda i: (i, 0)),
        out_shape=jax.ShapeDtypeStruct((N, H), jnp.float32),
    )(x, wm_pad, b, st)


def _ie_kernel(f_ref, wmi_ref, st_ref, o_ref):
    st = st_ref[...]
    rs = 1.0 / jnp.sqrt(st[1:2, :] + 1e-5)
    fn = f_ref[...] * rs
    o_ref[...] = jnp.dot(fn, wmi_ref[...], preferred_element_type=jnp.float32,
                 precision=lax.Precision.HIGHEST)


def _ie_mm(f, wmi, st):
    return pl.pallas_call(
        _ie_kernel,
        grid=(E // _CE_I,),
        in_specs=[pl.BlockSpec((_CE_I, 8), lambda i: (i, 0)),
                  pl.BlockSpec((8, H), lambda i: (0, 0)),
                  pl.BlockSpec((2, 8), lambda i: (0, 0))],
        out_specs=pl.BlockSpec((_CE_I, H), lambda i: (i, 0)),
        out_shape=jax.ShapeDtypeStruct((E, H), jnp.float32),
    )(f, wmi, st)


def _upd_res_kernel(p_ref, x_ref, w_ref, b_ref, o_ref):
    a = p_ref[...]
    h = jnp.dot(a, w_ref[...], preferred_element_type=jnp.float32,
                 precision=lax.Precision.HIGHEST) + b_ref[...]
    o_ref[...] = x_ref[...] + _silu(h)


def _upd_cross_kernel(p_ref, w_ref, b_ref, o_ref):
    a = p_ref[...]
    h = jnp.dot(a, w_ref[...], preferred_element_type=jnp.float32,
                 precision=lax.Precision.HIGHEST) + b_ref[...]
    o_ref[...] = _silu(h)


def _update(partials, x, w, b, residual):
    pspec = pl.BlockSpec((_BN, H), lambda i: (i, 0))
    wspec = pl.BlockSpec((H, H), lambda i: (0, 0))
    bspec = pl.BlockSpec((1, H), lambda i: (0, 0))
    ospec = pl.BlockSpec((_BN, H), lambda i: (i, 0))
    oshape = jax.ShapeDtypeStruct((N, H), jnp.float32)
    if residual:
        return pl.pallas_call(
            _upd_res_kernel, grid=(N // _BN,),
            in_specs=[pspec, ospec, wspec, bspec],
            out_specs=ospec, out_shape=oshape,
        )(partials, x, w, b)
    return pl.pallas_call(
        _upd_cross_kernel, grid=(N // _BN,),
        in_specs=[pspec, wspec, bspec],
        out_specs=ospec, out_shape=oshape,
    )(partials, w, b)


def _pool_kernel(b_ref, xa_ref, xb_ref, xc_ref, oa_ref, ob_ref, oc_ref):
    i = pl.program_id(0)
    b = b_ref[0]
    seg = lax.broadcasted_iota(jnp.int32, (NG, 1), 0)
    oh = (b == seg).astype(jnp.float32)
    for xr, orf in ((xa_ref, oa_ref), (xb_ref, ob_ref), (xc_ref, oc_ref)):
        blk = lax.dot_general(oh, xr[...], (((1,), (0,)), ((), ())),
                              preferred_element_type=jnp.float32,
                 precision=lax.Precision.HIGHEST)

        @pl.when(i == 0)
        def _():
            orf[...] = blk

        @pl.when(i > 0)
        def _():
            orf[...] = orf[...] + blk


def _pool(batch3, xa, xb, xc):
    xspec = pl.BlockSpec((_BN, H), lambda i: (i, 0))
    ospec = pl.BlockSpec((NG, H), lambda i: (0, 0))
    oshape = jax.ShapeDtypeStruct((NG, H), jnp.float32)
    return pl.pallas_call(
        _pool_kernel,
        grid=(N // _BN,),
        in_specs=[pl.BlockSpec((1, 1, _BN), lambda i: (i, 0, 0)),
                  xspec, xspec, xspec],
        out_specs=[ospec, ospec, ospec],
        out_shape=[oshape, oshape, oshape],
    )(batch3, xa, xb, xc)


def _head_kernel(pa_ref, pb_ref, pc_ref, w1_ref, b1_ref, w2_ref, b2_ref, o_ref):
    h = (jnp.dot(pa_ref[...], w1_ref[0:H, :], preferred_element_type=jnp.float32,
                 precision=lax.Precision.HIGHEST)
         + jnp.dot(pb_ref[...], w1_ref[H:2 * H, :], preferred_element_type=jnp.float32,
                 precision=lax.Precision.HIGHEST)
         + jnp.dot(pc_ref[...], w1_ref[2 * H:3 * H, :], preferred_element_type=jnp.float32,
                 precision=lax.Precision.HIGHEST)
         + b1_ref[...])
    h = _silu(h)
    o_ref[...] = jnp.dot(h, w2_ref[...], preferred_element_type=jnp.float32,
                 precision=lax.Precision.HIGHEST) + b2_ref[...]


def _head(pa, pb, pc, w1, b1, w2p, b2p):
    full = lambda shape: pl.BlockSpec(shape, lambda: tuple(0 for _ in shape))
    return pl.pallas_call(
        _head_kernel,
        in_specs=[full((NG, H)), full((NG, H)), full((NG, H)),
                  full((3 * H, H)), full((1, H)), full((H, 8)), full((1, 8))],
        out_specs=full((NG, 8)),
        out_shape=jax.ShapeDtypeStruct((NG, 8), jnp.float32),
    )(pa, pb, pc, w1, b1, w2p, b2p)


def kernel(x0, x1, x2, pos, params, edge_index_0, edge_index_1, edge_index_2, batch):
    del x1, x2
    e0 = edge_index_0.astype(jnp.int32)
    e1 = edge_index_1.astype(jnp.int32)
    e2 = edge_index_2.astype(jnp.int32)
    pos = pos.astype(jnp.float32)
    batch3 = batch.astype(jnp.int32).reshape(N // _BN, 1, _BN)
    pad = 2504 * _GP - E
    srcs = {0: jnp.pad(e0[0], (0, pad)), 1: jnp.pad(e1[0], (0, pad)),
            2: jnp.pad(e2[0], (0, pad))}
    dsts = {0: jnp.pad(e0[1], (0, pad)), 1: jnp.pad(e1[1], (0, pad)),
            2: jnp.pad(e2[1], (0, pad))}

    pos128 = jnp.pad(pos, ((0, 0), (0, H - 3)))
    inv_call, mp_call = _sc_kernels()
    dd0, dd1, dd2 = inv_call(
        pos128, srcs[0], dsts[0], srcs[1], dsts[1], srcs[2], dsts[2])
    fs = {0: _f_mm(dd0), 1: _f_mm(dd1), 2: _f_mm(dd2)}
    st = {0: _stats(fs[0]), 1: _stats(fs[1]), 2: _stats(fs[2])}

    def mp(x, r, p, wkey, bkey, residual):
        wm = jnp.pad(p['Wm'], ((0, 3), (0, 0)))
        y = _y_mm(x, wm, p['bm'][None, :], st[r])
        ie = _ie_mm(fs[r], wm[H:H + 8], st[r])
        partials = mp_call(y, ie, srcs[r], dsts[r])
        return _update(partials, x, p[wkey], p[bkey][None, :], residual)

    pe = params['emb']['0']
    x = _mm_bias(x0, pe['W'], pe['b'][None, :])
    for l in range(2):
        x = mp(x, 0, params['intra']['0_' + str(l)], 'Wu', 'bu', True)
    x1h = mp(x, 1, params['cross']['0_1'], 'Wo', 'bo', False)
    for l in range(2):
        x1h = mp(x1h, 1, params['intra']['1_' + str(l)], 'Wu', 'bu', True)
    x2h = mp(x1h, 2, params['cross']['1_2'], 'Wo', 'bo', False)
    for l in range(2):
        x2h = mp(x2h, 2, params['intra']['2_' + str(l)], 'Wu', 'bu', True)

    pa, pb, pc = _pool(batch3, x, x1h, x2h)
    pp = params['pool']
    w2p = jnp.pad(pp['W2'], ((0, 0), (0, 7)))
    b2p = jnp.pad(pp['b2'], (0, 7))[None, :]
    out = _head(pa, pb, pc, pp['W1'], pp['b1'][None, :], w2p, b2p)
    return out[:, 0:1]
